# Initial kernel scaffold; baseline (speedup 1.0000x reference)
import jax
import jax.numpy as jnp
from jax.experimental import pallas as pl


def kernel(x, edge_index, edge_attr, batch, W_emb, b_emb, Wf0, bf0, Ws0, bs0, gamma0, beta0, Wf1, bf1, Ws1, bs1, gamma1, beta1, Wf2, bf2, Ws2, bs2, gamma2, beta2, fc_W, fc_b, out_W, out_b):
    raise NotImplementedError("write your pallas kernel here")


# speedup vs baseline: 1.8026x; 1.8026x over previous
import jax
import jax.numpy as jnp
from jax import lax
from jax.experimental import pallas as pl
from jax.experimental.pallas import tpu as pltpu
from jax.experimental.pallas import tpu_sc as plsc

N = 50000
E = 800000
DF = 128
DE = 16
H = 64
G = 256

NTILE = 16
NCORE = 2
NW = NTILE * NCORE
NPAD = 50048
RPT = NPAD // NTILE

CG = 200
PER_W = E // NW
CS = 1000
PER_T = E // NTILE


def _gather_body(p_hbm, q_hbm, dst_hbm, src_hbm, outp_hbm, outq_hbm,
                 idx_d, idx_s, buf_p, buf_q, sem0, sem1):
    c = lax.axis_index("c")
    s = lax.axis_index("s")
    wid = s * NCORE + c
    base = wid * PER_W

    def chunk(i, carry):
        off = pl.multiple_of(base + i * CG, 8)
        pltpu.sync_copy(dst_hbm.at[pl.ds(off, CG)], idx_d)
        pltpu.sync_copy(src_hbm.at[pl.ds(off, CG)], idx_s)
        cp1 = pltpu.async_copy(p_hbm.at[idx_d], buf_p, sem0)
        cp2 = pltpu.async_copy(q_hbm.at[idx_s], buf_q, sem1)
        cp1.wait()
        cp2.wait()
        pltpu.sync_copy(buf_p, outp_hbm.at[pl.ds(off, CG)])
        pltpu.sync_copy(buf_q, outq_hbm.at[pl.ds(off, CG)])
        return carry

    lax.fori_loop(0, PER_W // CG, chunk, 0)


def _sc_gather(p, q, dst, src):
    mesh = plsc.VectorSubcoreMesh(core_axis_name="c", subcore_axis_name="s")
    f = pl.kernel(
        _gather_body,
        out_type=[jax.ShapeDtypeStruct((E, 2 * H), jnp.float32),
                  jax.ShapeDtypeStruct((E, 2 * H), jnp.float32)],
        mesh=mesh,
        scratch_types=[
            pltpu.VMEM((CG,), jnp.int32),
            pltpu.VMEM((CG,), jnp.int32),
            pltpu.VMEM((CG, 2 * H), jnp.float32),
            pltpu.VMEM((CG, 2 * H), jnp.float32),
            pltpu.SemaphoreType.DMA,
            pltpu.SemaphoreType.DMA,
        ],
    )
    return f(p, q, dst, src)


def _scatter_body(m_hbm, dst_hbm, out_hbm, ibuf, mbuf, acc):
    c = lax.axis_index("c")
    s = lax.axis_index("s")
    row0 = s * RPT
    base = s * PER_T
    z16 = jnp.zeros((16,), jnp.float32)

    def zrow(i, carry):
        mbuf[i, pl.ds(0, 16)] = z16
        return carry

    lax.fori_loop(0, CS, zrow, 0)

    for p in range(2):
        q = 2 * c + p
        for k in range(3):
            pltpu.sync_copy(mbuf, acc.at[pl.ds(row0 + k * CS, CS)])
        pltpu.sync_copy(mbuf.at[pl.ds(0, RPT - 3 * CS)],
                        acc.at[pl.ds(row0 + 3 * CS, RPT - 3 * CS)])
        plsc.subcore_barrier()

        def chunk(i, carry):
            off = pl.multiple_of(base + i * CS, 8)
            pltpu.sync_copy(m_hbm.at[q, pl.ds(off, CS)], mbuf)
            pltpu.sync_copy(dst_hbm.at[pl.ds(off, CS)], ibuf)
            pltpu.sync_copy(mbuf, acc.at[ibuf], add=True)
            return carry

        lax.fori_loop(0, PER_T // CS, chunk, 0)
        plsc.subcore_barrier()

        for k in range(3):
            r = row0 + k * CS
            pltpu.sync_copy(acc.at[pl.ds(r, CS)], mbuf)
            pltpu.sync_copy(mbuf, out_hbm.at[q, pl.ds(r, CS)])
        r = row0 + 3 * CS
        tail = RPT - 3 * CS
        pltpu.sync_copy(acc.at[pl.ds(r, tail)], mbuf.at[pl.ds(0, tail)])
        pltpu.sync_copy(mbuf.at[pl.ds(0, tail)], out_hbm.at[q, pl.ds(r, tail)])
        if p == 0:
            lax.fori_loop(0, CS, zrow, 0)


def _sc_scatter(m4, dst):
    mesh = plsc.VectorSubcoreMesh(core_axis_name="c", subcore_axis_name="s")
    f = pl.kernel(
        _scatter_body,
        out_type=jax.ShapeDtypeStruct((4, NPAD, DE), jnp.float32),
        mesh=mesh,
        compiler_params=pltpu.CompilerParams(use_tc_tiling_on_sc=False),
        scratch_types=[
            pltpu.VMEM((CS,), jnp.int32),
            pltpu.VMEM((CS, DE), jnp.float32),
            pltpu.VMEM_SHARED((NPAD, DE), jnp.float32),
        ],
    )
    return f(m4, dst)


def _embed_kernel(x_ref, w_ref, b_ref, o_ref):
    o_ref[...] = jnp.dot(x_ref[...], w_ref[...],
                         preferred_element_type=jnp.float32,
                         precision=lax.Precision.HIGHEST) + b_ref[...]


def _tc_embed(x, W_emb, b_emb):
    blk = 2000
    return pl.pallas_call(
        _embed_kernel,
        grid=(N // blk,),
        in_specs=[
            pl.BlockSpec((blk, DF), lambda i: (i, 0)),
            pl.BlockSpec((DF, H), lambda i: (0, 0)),
            pl.BlockSpec((1, H), lambda i: (0, 0)),
        ],
        out_specs=pl.BlockSpec((blk, H), lambda i: (i, 0)),
        out_shape=jax.ShapeDtypeStruct((N, H), jnp.float32),
    )(x, W_emb, b_emb.reshape(1, H))


def _proj_kernel(h_ref, ad_ref, as_ref, b_ref, p_ref, q_ref):
    h = h_ref[...]
    p_ref[...] = jnp.dot(h, ad_ref[...],
                         preferred_element_type=jnp.float32,
                         precision=lax.Precision.HIGHEST) + b_ref[...]
    q_ref[...] = jnp.dot(h, as_ref[...], preferred_element_type=jnp.float32,
                         precision=lax.Precision.HIGHEST)


def _tc_proj(h, A_d, A_s, bias):
    blk = 2000
    return pl.pallas_call(
        _proj_kernel,
        grid=(N // blk,),
        in_specs=[
            pl.BlockSpec((blk, H), lambda i: (i, 0)),
            pl.BlockSpec((H, 2 * H), lambda i: (0, 0)),
            pl.BlockSpec((H, 2 * H), lambda i: (0, 0)),
            pl.BlockSpec((1, 2 * H), lambda i: (0, 0)),
        ],
        out_specs=[pl.BlockSpec((blk, 2 * H), lambda i: (i, 0)),
                   pl.BlockSpec((blk, 2 * H), lambda i: (i, 0))],
        out_shape=[jax.ShapeDtypeStruct((N, 2 * H), jnp.float32),
                   jax.ShapeDtypeStruct((N, 2 * H), jnp.float32)],
    )(h, A_d, A_s, bias)


def _message_kernel(pd_ref, qs_ref, ea_ref, ae_ref, o_ref):
    t = pd_ref[...] + qs_ref[...]
    t += lax.dot_general(ea_ref[...], ae_ref[...],
                         (((0,), (0,)), ((), ())),
                         preferred_element_type=jnp.float32,
                         precision=lax.Precision.HIGHEST)
    tf = t[:, :H]
    ts = t[:, H:]
    sig = 1.0 / (1.0 + jnp.exp(-tf))
    sp = jnp.log(1.0 + jnp.exp(-jnp.abs(ts))) + jnp.maximum(ts, 0.0)
    m = sig * sp
    for qq in range(4):
        o_ref[qq] = m[:, DE * qq:DE * (qq + 1)]


def _tc_message(pd, qs, ea_t, A_e):
    blk = 3200
    return pl.pallas_call(
        _message_kernel,
        grid=(E // blk,),
        in_specs=[
            pl.BlockSpec((blk, 2 * H), lambda i: (i, 0)),
            pl.BlockSpec((blk, 2 * H), lambda i: (i, 0)),
            pl.BlockSpec((DE, blk), lambda i: (0, i)),
            pl.BlockSpec((DE, 2 * H), lambda i: (0, 0)),
        ],
        out_specs=pl.BlockSpec((4, blk, DE), lambda i: (0, i, 0)),
        out_shape=jax.ShapeDtypeStruct((4, E, DE), jnp.float32),
    )(pd, qs, ea_t, A_e)


def _bnstats_kernel(a_ref, o_ref):
    @pl.when(pl.program_id(0) == 0)
    def _():
        o_ref[...] = jnp.zeros_like(o_ref)

    s = jnp.concatenate(
        [jnp.sum(a_ref[qq], axis=0, keepdims=True) for qq in range(4)], axis=1)
    o_ref[0:1, :] += s


def _tc_bnstats(aggp):
    blk = RPT
    return pl.pallas_call(
        _bnstats_kernel,
        grid=(NPAD // blk,),
        in_specs=[pl.BlockSpec((4, blk, DE), lambda i: (0, i, 0))],
        out_specs=pl.BlockSpec((1, H), lambda i: (0, 0)),
        out_shape=jax.ShapeDtypeStruct((1, H), jnp.float32),
    )(aggp)


def _bnvar_kernel(a_ref, s_ref, o_ref):
    @pl.when(pl.program_id(0) == 0)
    def _():
        o_ref[...] = jnp.zeros_like(o_ref)

    mu = s_ref[...] * (1.0 / N)
    sq = []
    for qq in range(4):
        d = a_ref[qq] - mu[:, DE * qq:DE * (qq + 1)]
        sq.append(jnp.sum(d * d, axis=0, keepdims=True))
    o_ref[...] += jnp.concatenate(sq, axis=1)


def _tc_bnvar(aggp, stats):
    blk = RPT
    return pl.pallas_call(
        _bnvar_kernel,
        grid=(NPAD // blk,),
        in_specs=[pl.BlockSpec((4, blk, DE), lambda i: (0, i, 0)),
                  pl.BlockSpec((1, H), lambda i: (0, 0))],
        out_specs=pl.BlockSpec((1, H), lambda i: (0, 0)),
        out_shape=jax.ShapeDtypeStruct((1, H), jnp.float32),
    )(aggp, stats)


def _bnapply_kernel(a_ref, h_ref, st_ref, sv_ref, g_ref, be_ref, o_ref):
    mu = st_ref[...] * (1.0 / N)
    var = (sv_ref[...] - (NPAD - N) * mu * mu) * (1.0 / N)
    inv = lax.rsqrt(var + 1e-5)
    agg = jnp.concatenate([a_ref[qq] for qq in range(4)], axis=1)
    o_ref[...] = g_ref[...] * (agg - mu) * inv + be_ref[...] + h_ref[...]


def _tc_bnapply(aggp, h, stats, statv, gamma, beta):
    blk = 2000
    return pl.pallas_call(
        _bnapply_kernel,
        grid=(N // blk,),
        in_specs=[
            pl.BlockSpec((4, blk, DE), lambda i: (0, i, 0)),
            pl.BlockSpec((blk, H), lambda i: (i, 0)),
            pl.BlockSpec((1, H), lambda i: (0, 0)),
            pl.BlockSpec((1, H), lambda i: (0, 0)),
            pl.BlockSpec((1, H), lambda i: (0, 0)),
            pl.BlockSpec((1, H), lambda i: (0, 0)),
        ],
        out_specs=pl.BlockSpec((blk, H), lambda i: (i, 0)),
        out_shape=jax.ShapeDtypeStruct((N, H), jnp.float32),
    )(aggp, h, stats, statv, gamma.reshape(1, H), beta.reshape(1, H))


def _pool_kernel(h_ref, b_ref, s_ref, c_ref):
    @pl.when(pl.program_id(0) == 0)
    def _():
        s_ref[...] = jnp.zeros_like(s_ref)
        c_ref[...] = jnp.zeros_like(c_ref)

    blk = h_ref.shape[0]
    b = b_ref[0, 0, :]
    oh = (b[:, None] == lax.broadcasted_iota(jnp.int32, (blk, G), 1))
    oh = oh.astype(jnp.float32)
    s_ref[...] += lax.dot_general(oh, h_ref[...], (((0,), (0,)), ((), ())),
                                  preferred_element_type=jnp.float32,
                         precision=lax.Precision.HIGHEST)
    c_ref[...] += lax.dot_general(oh, jnp.ones((blk, 8), jnp.float32),
                                  (((0,), (0,)), ((), ())),
                                  preferred_element_type=jnp.float32,
                         precision=lax.Precision.HIGHEST)


def _tc_pool(h, batch3d):
    blk = 2000
    return pl.pallas_call(
        _pool_kernel,
        grid=(N // blk,),
        in_specs=[
            pl.BlockSpec((blk, H), lambda i: (i, 0)),
            pl.BlockSpec((1, 1, blk), lambda i: (i, 0, 0)),
        ],
        out_specs=[
            pl.BlockSpec((G, H), lambda i: (0, 0)),
            pl.BlockSpec((G, 8), lambda i: (0, 0)),
        ],
        out_shape=[jax.ShapeDtypeStruct((G, H), jnp.float32),
                   jax.ShapeDtypeStruct((G, 8), jnp.float32)],
    )(h, batch3d)


def _head_kernel(s_ref, c_ref, fw_ref, fb_ref, ow_ref, ob_ref, o_ref):
    pooled = s_ref[...] / jnp.maximum(c_ref[:, 0:1], 1.0)
    t = jnp.dot(pooled, fw_ref[...], preferred_element_type=jnp.float32,
                         precision=lax.Precision.HIGHEST)
    t += fb_ref[...]
    hfc = jnp.log(1.0 + jnp.exp(-jnp.abs(t))) + jnp.maximum(t, 0.0)
    res = jnp.sum(hfc * ow_ref[...], axis=1) + ob_ref[0, 0]
    o_ref[...] = jnp.broadcast_to(res[None, :], (8, G))


def _tc_head(sums, counts, fc_W, fc_b, out_W, out_b):
    return pl.pallas_call(
        _head_kernel,
        grid=(1,),
        in_specs=[
            pl.BlockSpec((G, H), lambda i: (0, 0)),
            pl.BlockSpec((G, 8), lambda i: (0, 0)),
            pl.BlockSpec((H, H), lambda i: (0, 0)),
            pl.BlockSpec((1, H), lambda i: (0, 0)),
            pl.BlockSpec((1, H), lambda i: (0, 0)),
            pl.BlockSpec((1, 1), lambda i: (0, 0)),
        ],
        out_specs=pl.BlockSpec((8, G), lambda i: (0, 0)),
        out_shape=jax.ShapeDtypeStruct((8, G), jnp.float32),
    )(sums, counts, fc_W, fc_b.reshape(1, H), out_W.reshape(1, H),
      out_b.reshape(1, 1))


def kernel(x, edge_index, edge_attr, batch, W_emb, b_emb,
           Wf0, bf0, Ws0, bs0, gamma0, beta0,
           Wf1, bf1, Ws1, bs1, gamma1, beta1,
           Wf2, bf2, Ws2, bs2, gamma2, beta2,
           fc_W, fc_b, out_W, out_b):
    src = edge_index[0]
    dst = edge_index[1]
    ea_t = edge_attr.T
    batch3d = batch.reshape(N // 2000, 1, 2000)

    h = _tc_embed(x, W_emb, b_emb)

    for (Wf, bf, Ws, bs, gamma, beta) in (
            (Wf0, bf0, Ws0, bs0, gamma0, beta0),
            (Wf1, bf1, Ws1, bs1, gamma1, beta1),
            (Wf2, bf2, Ws2, bs2, gamma2, beta2)):
        A_d = jnp.concatenate([Wf[:H], Ws[:H]], axis=1)
        A_s = jnp.concatenate([Wf[H:2 * H], Ws[H:2 * H]], axis=1)
        A_e = jnp.concatenate([Wf[2 * H:], Ws[2 * H:]], axis=1)
        bias = jnp.concatenate([bf, bs]).reshape(1, 2 * H)

        p, q = _tc_proj(h, A_d, A_s, bias)
        pd, qs = _sc_gather(p, q, dst, src)
        m4 = _tc_message(pd, qs, ea_t, A_e)
        aggp = _sc_scatter(m4, dst)
        stats = _tc_bnstats(aggp)
        statv = _tc_bnvar(aggp, stats)
        h = _tc_bnapply(aggp, h, stats, statv, gamma, beta)

    sums, counts = _tc_pool(h, batch3d)
    out2d = _tc_head(sums, counts, fc_W, fc_b, out_W, out_b)
    return out2d[0]


# speedup vs baseline: 1.9266x; 1.0688x over previous
import jax
import jax.numpy as jnp
from jax import lax
from jax.experimental import pallas as pl
from jax.experimental.pallas import tpu as pltpu
from jax.experimental.pallas import tpu_sc as plsc

N = 50000
E = 800000
DF = 128
DE = 16
H = 64
G = 256

NTILE = 16
NCORE = 2
NW = NTILE * NCORE
NPAD = 50048
RPT = NPAD // NTILE

CG = 200
PER_W = E // NW
CS = 1000
PER_T = E // NTILE


IGRP = 2000
GCH = IGRP // CG
NGRP = PER_W // IGRP


def _gather_body(p_hbm, q_hbm, dst_hbm, src_hbm, outp_hbm, outq_hbm,
                 idx_d, idx_s, bp0, bp1, bq0, bq1, sp0, sp1, sq0, sq1):
    c = lax.axis_index("c")
    s = lax.axis_index("s")
    wid = s * NCORE + c
    base = wid * PER_W
    bufp = (bp0, bp1)
    bufq = (bq0, bq1)
    semp = (sp0, sp1)
    semq = (sq0, sq1)

    def run_group(goff, nch):
        pltpu.sync_copy(dst_hbm.at[pl.ds(goff, nch * CG)],
                        idx_d.at[pl.ds(0, nch * CG)])
        pltpu.sync_copy(src_hbm.at[pl.ds(goff, nch * CG)],
                        idx_s.at[pl.ds(0, nch * CG)])
        cps = [None, None]
        for k in range(nch):
            st = k % 2
            if cps[st] is not None:
                cp1, cp2, off_prev = cps[st]
                cp1.wait()
                cp2.wait()
                pltpu.sync_copy(bufp[st], outp_hbm.at[pl.ds(off_prev, CG)])
                pltpu.sync_copy(bufq[st], outq_hbm.at[pl.ds(off_prev, CG)])
            off = pl.multiple_of(goff + k * CG, 8)
            cp1 = pltpu.async_copy(p_hbm.at[idx_d.at[pl.ds(k * CG, CG)]],
                                   bufp[st], semp[st])
            cp2 = pltpu.async_copy(q_hbm.at[idx_s.at[pl.ds(k * CG, CG)]],
                                   bufq[st], semq[st])
            cps[st] = (cp1, cp2, off)
        for st in ((nch % 2), (nch + 1) % 2):
            cp1, cp2, off_prev = cps[st]
            cp1.wait()
            cp2.wait()
            pltpu.sync_copy(bufp[st], outp_hbm.at[pl.ds(off_prev, CG)])
            pltpu.sync_copy(bufq[st], outq_hbm.at[pl.ds(off_prev, CG)])

    def group(g, carry):
        run_group(pl.multiple_of(base + g * IGRP, 8), GCH)
        return carry

    lax.fori_loop(0, NGRP, group, 0)
    run_group(pl.multiple_of(base + NGRP * IGRP, 8),
              (PER_W - NGRP * IGRP) // CG)


def _sc_gather(p, q, dst, src):
    mesh = plsc.VectorSubcoreMesh(core_axis_name="c", subcore_axis_name="s")
    f = pl.kernel(
        _gather_body,
        out_type=[jax.ShapeDtypeStruct((E, 2 * H), jnp.float32),
                  jax.ShapeDtypeStruct((E, 2 * H), jnp.float32)],
        mesh=mesh,
        scratch_types=[
            pltpu.VMEM((IGRP,), jnp.int32),
            pltpu.VMEM((IGRP,), jnp.int32),
            pltpu.VMEM((CG, 2 * H), jnp.float32),
            pltpu.VMEM((CG, 2 * H), jnp.float32),
            pltpu.VMEM((CG, 2 * H), jnp.float32),
            pltpu.VMEM((CG, 2 * H), jnp.float32),
            pltpu.SemaphoreType.DMA,
            pltpu.SemaphoreType.DMA,
            pltpu.SemaphoreType.DMA,
            pltpu.SemaphoreType.DMA,
        ],
    )
    return f(p, q, dst, src)


def _scatter_body(m_hbm, dst_hbm, out_hbm, ibuf, mbuf, acc):
    c = lax.axis_index("c")
    s = lax.axis_index("s")
    row0 = s * RPT
    base = s * PER_T
    z16 = jnp.zeros((16,), jnp.float32)

    def zrow(i, carry):
        mbuf[i, pl.ds(0, 16)] = z16
        return carry

    lax.fori_loop(0, CS, zrow, 0)

    for p in range(2):
        q = 2 * c + p
        for k in range(3):
            pltpu.sync_copy(mbuf, acc.at[pl.ds(row0 + k * CS, CS)])
        pltpu.sync_copy(mbuf.at[pl.ds(0, RPT - 3 * CS)],
                        acc.at[pl.ds(row0 + 3 * CS, RPT - 3 * CS)])
        plsc.subcore_barrier()

        def chunk(i, carry):
            off = pl.multiple_of(base + i * CS, 8)
            pltpu.sync_copy(m_hbm.at[q, pl.ds(off, CS)], mbuf)
            pltpu.sync_copy(dst_hbm.at[pl.ds(off, CS)], ibuf)
            pltpu.sync_copy(mbuf, acc.at[ibuf], add=True)
            return carry

        lax.fori_loop(0, PER_T // CS, chunk, 0)
        plsc.subcore_barrier()

        for k in range(3):
            r = row0 + k * CS
            pltpu.sync_copy(acc.at[pl.ds(r, CS)], mbuf)
            pltpu.sync_copy(mbuf, out_hbm.at[q, pl.ds(r, CS)])
        r = row0 + 3 * CS
        tail = RPT - 3 * CS
        pltpu.sync_copy(acc.at[pl.ds(r, tail)], mbuf.at[pl.ds(0, tail)])
        pltpu.sync_copy(mbuf.at[pl.ds(0, tail)], out_hbm.at[q, pl.ds(r, tail)])
        if p == 0:
            lax.fori_loop(0, CS, zrow, 0)


def _sc_scatter(m4, dst):
    mesh = plsc.VectorSubcoreMesh(core_axis_name="c", subcore_axis_name="s")
    f = pl.kernel(
        _scatter_body,
        out_type=jax.ShapeDtypeStruct((4, NPAD, DE), jnp.float32),
        mesh=mesh,
        compiler_params=pltpu.CompilerParams(use_tc_tiling_on_sc=False),
        scratch_types=[
            pltpu.VMEM((CS,), jnp.int32),
            pltpu.VMEM((CS, DE), jnp.float32),
            pltpu.VMEM_SHARED((NPAD, DE), jnp.float32),
        ],
    )
    return f(m4, dst)


def _embed_proj_kernel(x_ref, w_ref, b_ref, ad_ref, as_ref, bb_ref,
                       h_ref, p_ref, q_ref):
    h = jnp.dot(x_ref[...], w_ref[...],
                preferred_element_type=jnp.float32,
                precision=lax.Precision.HIGHEST) + b_ref[...]
    h_ref[...] = h
    p_ref[...] = jnp.dot(h, ad_ref[...],
                         preferred_element_type=jnp.float32,
                         precision=lax.Precision.HIGHEST) + bb_ref[...]
    q_ref[...] = jnp.dot(h, as_ref[...], preferred_element_type=jnp.float32,
                         precision=lax.Precision.HIGHEST)


def _tc_embed_proj(x, W_emb, b_emb, A_d, A_s, bias):
    blk = 2000
    return pl.pallas_call(
        _embed_proj_kernel,
        grid=(N // blk,),
        in_specs=[
            pl.BlockSpec((blk, DF), lambda i: (i, 0)),
            pl.BlockSpec((DF, H), lambda i: (0, 0)),
            pl.BlockSpec((1, H), lambda i: (0, 0)),
            pl.BlockSpec((H, 2 * H), lambda i: (0, 0)),
            pl.BlockSpec((H, 2 * H), lambda i: (0, 0)),
            pl.BlockSpec((1, 2 * H), lambda i: (0, 0)),
        ],
        out_specs=[pl.BlockSpec((blk, H), lambda i: (i, 0)),
                   pl.BlockSpec((blk, 2 * H), lambda i: (i, 0)),
                   pl.BlockSpec((blk, 2 * H), lambda i: (i, 0))],
        out_shape=[jax.ShapeDtypeStruct((N, H), jnp.float32),
                   jax.ShapeDtypeStruct((N, 2 * H), jnp.float32),
                   jax.ShapeDtypeStruct((N, 2 * H), jnp.float32)],
    )(x, W_emb, b_emb.reshape(1, H), A_d, A_s, bias)


def _message_kernel(pd_ref, qs_ref, ea_ref, ae_ref, o_ref):
    t = pd_ref[...] + qs_ref[...]
    t += lax.dot_general(ea_ref[...], ae_ref[...],
                         (((0,), (0,)), ((), ())),
                         preferred_element_type=jnp.float32,
                         precision=lax.Precision.HIGHEST)
    tf = t[:, :H]
    ts = t[:, H:]
    sig = 1.0 / (1.0 + jnp.exp(-tf))
    sp = jnp.log(1.0 + jnp.exp(-jnp.abs(ts))) + jnp.maximum(ts, 0.0)
    m = sig * sp
    for qq in range(4):
        o_ref[qq] = m[:, DE * qq:DE * (qq + 1)]


def _tc_message(pd, qs, ea_t, A_e):
    blk = 3200
    return pl.pallas_call(
        _message_kernel,
        grid=(E // blk,),
        in_specs=[
            pl.BlockSpec((blk, 2 * H), lambda i: (i, 0)),
            pl.BlockSpec((blk, 2 * H), lambda i: (i, 0)),
            pl.BlockSpec((DE, blk), lambda i: (0, i)),
            pl.BlockSpec((DE, 2 * H), lambda i: (0, 0)),
        ],
        out_specs=pl.BlockSpec((4, blk, DE), lambda i: (0, i, 0)),
        out_shape=jax.ShapeDtypeStruct((4, E, DE), jnp.float32),
    )(pd, qs, ea_t, A_e)


def _bnstats_kernel(a_ref, o_ref):
    @pl.when(pl.program_id(0) == 0)
    def _():
        o_ref[...] = jnp.zeros_like(o_ref)

    s = jnp.concatenate(
        [jnp.sum(a_ref[qq], axis=0, keepdims=True) for qq in range(4)], axis=1)
    o_ref[0:1, :] += s


def _tc_bnstats(aggp):
    blk = RPT
    return pl.pallas_call(
        _bnstats_kernel,
        grid=(NPAD // blk,),
        in_specs=[pl.BlockSpec((4, blk, DE), lambda i: (0, i, 0))],
        out_specs=pl.BlockSpec((1, H), lambda i: (0, 0)),
        out_shape=jax.ShapeDtypeStruct((1, H), jnp.float32),
    )(aggp)


def _bnvar_kernel(a_ref, s_ref, o_ref):
    @pl.when(pl.program_id(0) == 0)
    def _():
        o_ref[...] = jnp.zeros_like(o_ref)

    mu = s_ref[...] * (1.0 / N)
    sq = []
    for qq in range(4):
        d = a_ref[qq] - mu[:, DE * qq:DE * (qq + 1)]
        sq.append(jnp.sum(d * d, axis=0, keepdims=True))
    o_ref[...] += jnp.concatenate(sq, axis=1)


def _tc_bnvar(aggp, stats):
    blk = RPT
    return pl.pallas_call(
        _bnvar_kernel,
        grid=(NPAD // blk,),
        in_specs=[pl.BlockSpec((4, blk, DE), lambda i: (0, i, 0)),
                  pl.BlockSpec((1, H), lambda i: (0, 0))],
        out_specs=pl.BlockSpec((1, H), lambda i: (0, 0)),
        out_shape=jax.ShapeDtypeStruct((1, H), jnp.float32),
    )(aggp, stats)


def _bn_update(a_ref, h_ref, st_ref, sv_ref, g_ref, be_ref):
    mu = st_ref[...] * (1.0 / N)
    var = (sv_ref[...] - (NPAD - N) * mu * mu) * (1.0 / N)
    inv = lax.rsqrt(var + 1e-5)
    agg = jnp.concatenate([a_ref[qq] for qq in range(4)], axis=1)
    return g_ref[...] * (agg - mu) * inv + be_ref[...] + h_ref[...]


def _apply_proj_kernel(a_ref, h_ref, st_ref, sv_ref, g_ref, be_ref,
                       ad_ref, as_ref, bb_ref, o_ref, p_ref, q_ref):
    h = _bn_update(a_ref, h_ref, st_ref, sv_ref, g_ref, be_ref)
    o_ref[...] = h
    p_ref[...] = jnp.dot(h, ad_ref[...],
                         preferred_element_type=jnp.float32,
                         precision=lax.Precision.HIGHEST) + bb_ref[...]
    q_ref[...] = jnp.dot(h, as_ref[...], preferred_element_type=jnp.float32,
                         precision=lax.Precision.HIGHEST)


def _tc_apply_proj(aggp, h, stats, statv, gamma, beta, A_d, A_s, bias):
    blk = 2000
    return pl.pallas_call(
        _apply_proj_kernel,
        grid=(N // blk,),
        in_specs=[
            pl.BlockSpec((4, blk, DE), lambda i: (0, i, 0)),
            pl.BlockSpec((blk, H), lambda i: (i, 0)),
            pl.BlockSpec((1, H), lambda i: (0, 0)),
            pl.BlockSpec((1, H), lambda i: (0, 0)),
            pl.BlockSpec((1, H), lambda i: (0, 0)),
            pl.BlockSpec((1, H), lambda i: (0, 0)),
            pl.BlockSpec((H, 2 * H), lambda i: (0, 0)),
            pl.BlockSpec((H, 2 * H), lambda i: (0, 0)),
            pl.BlockSpec((1, 2 * H), lambda i: (0, 0)),
        ],
        out_specs=[pl.BlockSpec((blk, H), lambda i: (i, 0)),
                   pl.BlockSpec((blk, 2 * H), lambda i: (i, 0)),
                   pl.BlockSpec((blk, 2 * H), lambda i: (i, 0))],
        out_shape=[jax.ShapeDtypeStruct((N, H), jnp.float32),
                   jax.ShapeDtypeStruct((N, 2 * H), jnp.float32),
                   jax.ShapeDtypeStruct((N, 2 * H), jnp.float32)],
    )(aggp, h, stats, statv, gamma.reshape(1, H), beta.reshape(1, H),
      A_d, A_s, bias)


def _apply_pool_kernel(a_ref, h_ref, st_ref, sv_ref, g_ref, be_ref, b_ref,
                       s_ref, c_ref):
    @pl.when(pl.program_id(0) == 0)
    def _():
        s_ref[...] = jnp.zeros_like(s_ref)
        c_ref[...] = jnp.zeros_like(c_ref)

    h = _bn_update(a_ref, h_ref, st_ref, sv_ref, g_ref, be_ref)
    blk = h.shape[0]
    b = b_ref[0, 0, :]
    oh = (b[:, None] == lax.broadcasted_iota(jnp.int32, (blk, G), 1))
    oh = oh.astype(jnp.float32)
    s_ref[...] += lax.dot_general(oh, h, (((0,), (0,)), ((), ())),
                                  preferred_element_type=jnp.float32,
                                  precision=lax.Precision.HIGHEST)
    c_ref[...] += lax.dot_general(oh, jnp.ones((blk, 8), jnp.float32),
                                  (((0,), (0,)), ((), ())),
                                  preferred_element_type=jnp.float32,
                                  precision=lax.Precision.HIGHEST)


def _tc_apply_pool(aggp, h, stats, statv, gamma, beta, batch3d):
    blk = 2000
    return pl.pallas_call(
        _apply_pool_kernel,
        grid=(N // blk,),
        in_specs=[
            pl.BlockSpec((4, blk, DE), lambda i: (0, i, 0)),
            pl.BlockSpec((blk, H), lambda i: (i, 0)),
            pl.BlockSpec((1, H), lambda i: (0, 0)),
            pl.BlockSpec((1, H), lambda i: (0, 0)),
            pl.BlockSpec((1, H), lambda i: (0, 0)),
            pl.BlockSpec((1, H), lambda i: (0, 0)),
            pl.BlockSpec((1, 1, blk), lambda i: (i, 0, 0)),
        ],
        out_specs=[
            pl.BlockSpec((G, H), lambda i: (0, 0)),
            pl.BlockSpec((G, 8), lambda i: (0, 0)),
        ],
        out_shape=[jax.ShapeDtypeStruct((G, H), jnp.float32),
                   jax.ShapeDtypeStruct((G, 8), jnp.float32)],
    )(aggp, h, stats, statv, gamma.reshape(1, H), beta.reshape(1, H), batch3d)


def _head_kernel(s_ref, c_ref, fw_ref, fb_ref, ow_ref, ob_ref, o_ref):
    pooled = s_ref[...] / jnp.maximum(c_ref[:, 0:1], 1.0)
    t = jnp.dot(pooled, fw_ref[...], preferred_element_type=jnp.float32,
                         precision=lax.Precision.HIGHEST)
    t += fb_ref[...]
    hfc = jnp.log(1.0 + jnp.exp(-jnp.abs(t))) + jnp.maximum(t, 0.0)
    res = jnp.sum(hfc * ow_ref[...], axis=1) + ob_ref[0, 0]
    o_ref[...] = jnp.broadcast_to(res[None, :], (8, G))


def _tc_head(sums, counts, fc_W, fc_b, out_W, out_b):
    return pl.pallas_call(
        _head_kernel,
        grid=(1,),
        in_specs=[
            pl.BlockSpec((G, H), lambda i: (0, 0)),
            pl.BlockSpec((G, 8), lambda i: (0, 0)),
            pl.BlockSpec((H, H), lambda i: (0, 0)),
            pl.BlockSpec((1, H), lambda i: (0, 0)),
            pl.BlockSpec((1, H), lambda i: (0, 0)),
            pl.BlockSpec((1, 1), lambda i: (0, 0)),
        ],
        out_specs=pl.BlockSpec((8, G), lambda i: (0, 0)),
        out_shape=jax.ShapeDtypeStruct((8, G), jnp.float32),
    )(sums, counts, fc_W, fc_b.reshape(1, H), out_W.reshape(1, H),
      out_b.reshape(1, 1))


def kernel(x, edge_index, edge_attr, batch, W_emb, b_emb,
           Wf0, bf0, Ws0, bs0, gamma0, beta0,
           Wf1, bf1, Ws1, bs1, gamma1, beta1,
           Wf2, bf2, Ws2, bs2, gamma2, beta2,
           fc_W, fc_b, out_W, out_b):
    src = edge_index[0]
    dst = edge_index[1]
    ea_t = edge_attr.T
    batch3d = batch.reshape(N // 2000, 1, 2000)

    convs = []
    for (Wf, bf, Ws, bs, gamma, beta) in (
            (Wf0, bf0, Ws0, bs0, gamma0, beta0),
            (Wf1, bf1, Ws1, bs1, gamma1, beta1),
            (Wf2, bf2, Ws2, bs2, gamma2, beta2)):
        A_d = jnp.concatenate([Wf[:H], Ws[:H]], axis=1)
        A_s = jnp.concatenate([Wf[H:2 * H], Ws[H:2 * H]], axis=1)
        A_e = jnp.concatenate([Wf[2 * H:], Ws[2 * H:]], axis=1)
        bias = jnp.concatenate([bf, bs]).reshape(1, 2 * H)
        convs.append((A_d, A_s, A_e, bias, gamma, beta))

    h, p, q = _tc_embed_proj(x, W_emb, b_emb, convs[0][0], convs[0][1],
                             convs[0][3])
    for i in range(3):
        A_d, A_s, A_e, bias, gamma, beta = convs[i]
        pd, qs = _sc_gather(p, q, dst, src)
        m4 = _tc_message(pd, qs, ea_t, A_e)
        aggp = _sc_scatter(m4, dst)
        stats = _tc_bnstats(aggp)
        statv = _tc_bnvar(aggp, stats)
        if i < 2:
            h, p, q = _tc_apply_proj(aggp, h, stats, statv, gamma, beta,
                                     convs[i + 1][0], convs[i + 1][1],
                                     convs[i + 1][3])
        else:
            sums, counts = _tc_apply_pool(aggp, h, stats, statv, gamma, beta,
                                          batch3d)

    out2d = _tc_head(sums, counts, fc_W, fc_b, out_W, out_b)
    return out2d[0]
